# baseline (speedup 1.0000x reference)
import functools

import jax
import jax.numpy as jnp
from jax import lax
from jax.experimental import pallas as pl
from jax.experimental.pallas import tpu as pltpu
from jax.experimental.pallas import tpu_sc as plsc

D = 32
BLK = 128
NC, NS = 2, 16
NW = NC * NS
NSLAB = D // 8


@functools.lru_cache(maxsize=None)
def _make_gather(F: int, Bt: int, V: int):
  n_blocks = F * (Bt // BLK)
  assert n_blocks % NW == 0
  j_per_w = n_blocks // NW
  nbh = Bt // BLK
  mesh = plsc.VectorSubcoreMesh(core_axis_name="c", subcore_axis_name="s")

  @functools.partial(
      pl.kernel,
      out_type=jax.ShapeDtypeStruct((F, NSLAB, nbh, 8, BLK), jnp.float32),
      mesh=mesh,
      compiler_params=pltpu.CompilerParams(
          use_tc_tiling_on_sc=False, needs_layout_passes=False),
      scratch_types=[
          pltpu.VMEM((j_per_w * BLK,), jnp.int32),
          pltpu.VMEM((2, BLK, D), jnp.float32),
          pltpu.VMEM((2, NSLAB, 8, BLK), jnp.float32),
          pltpu.SemaphoreType.DMA,
          pltpu.SemaphoreType.DMA,
      ],
  )
  def k(idx_hbm, table_hbm, out_hbm, idx_v, rows_v, t_v, gsem, psem):
    wid = lax.axis_index("s") * NC + lax.axis_index("c")
    g0 = wid * j_per_w

    def blk_of(j):
      g = g0 + j
      f = g // nbh
      bh = g - f * nbh
      return f, bh

    def start_gather(j, slot):
      return pltpu.async_copy(
          table_hbm.at[idx_v.at[pl.ds(j * BLK, BLK)]],
          rows_v.at[slot], gsem)

    def wait_gather(j, slot):
      pltpu.make_async_copy(
          table_hbm.at[idx_v.at[pl.ds(j * BLK, BLK)]],
          rows_v.at[slot], gsem).wait()

    def start_puts(j, slot):
      f, bh = blk_of(j)
      for si in range(NSLAB):
        pltpu.async_copy(t_v.at[slot, si], out_hbm.at[f, si, bh], psem)

    def wait_puts(j, slot):
      f, bh = blk_of(j)
      for si in range(NSLAB):
        pltpu.make_async_copy(
            t_v.at[slot, si], out_hbm.at[f, si, bh], psem).wait()

    lanes = lax.iota(jnp.int32, 16)

    def transpose(slot):
      src = rows_v.at[slot]
      for c in range(D):
        si, cl = c // 8, c % 8
        cvec = jnp.full((16,), c, jnp.int32)
        for chunk in range(BLK // 16):
          bl = lanes + (chunk * 16)
          vec = plsc.load_gather(src, [bl, cvec])
          t_v[slot, si, cl, pl.ds(chunk * 16, 16)] = vec

    pltpu.sync_copy(idx_hbm.at[pl.ds(g0 * BLK, j_per_w * BLK)], idx_v)
    start_gather(0, 0)

    def body(j, _):
      s = lax.rem(j, 2)
      p = 1 - s
      wait_gather(j - 1, p)
      start_gather(j, s)

      @pl.when(j >= 2)
      def _():
        wait_puts(j - 2, p)

      transpose(p)
      start_puts(j - 1, p)
      return 0

    lax.fori_loop(1, j_per_w, body, 0)

    last = j_per_w - 1
    sl = last % 2
    wait_gather(last, sl)
    transpose(sl)
    start_puts(last, sl)
    wait_puts(last - 1, 1 - sl)
    wait_puts(last, sl)

  return k


def kernel(x, table):
  Bt, F = x.shape
  V, d = table.shape
  assert d == D and Bt % BLK == 0
  idx = jnp.swapaxes(x, 0, 1).reshape(F * Bt).astype(jnp.int32)
  out5 = _make_gather(F, Bt, V)(idx, table)
  return out5.transpose(2, 4, 0, 1, 3).reshape(Bt, F, D)


# speedup vs baseline: 1.2400x; 1.2400x over previous
import functools

import jax
import jax.numpy as jnp
from jax import lax
from jax.experimental import pallas as pl
from jax.experimental.pallas import tpu as pltpu
from jax.experimental.pallas import tpu_sc as plsc

D = 32
BLK = 128
NC, NS = 2, 16
NW = NC * NS
NSLAB = D // 8


@functools.lru_cache(maxsize=None)
def _make_gather(F: int, Bt: int, V: int):
  n_blocks = F * (Bt // BLK)
  assert n_blocks % NW == 0
  j_per_w = n_blocks // NW
  nbh = Bt // BLK
  mesh = plsc.VectorSubcoreMesh(core_axis_name="c", subcore_axis_name="s")

  @functools.partial(
      pl.kernel,
      out_type=jax.ShapeDtypeStruct((F, NSLAB, nbh, 8, BLK), jnp.float32),
      mesh=mesh,
      compiler_params=pltpu.CompilerParams(
          use_tc_tiling_on_sc=False, needs_layout_passes=False),
      scratch_types=[
          pltpu.VMEM((j_per_w * BLK,), jnp.int32),
          pltpu.VMEM((2, BLK, D), jnp.float32),
          pltpu.VMEM((2, D, BLK), jnp.float32),
          pltpu.SemaphoreType.DMA,
          pltpu.SemaphoreType.DMA,
      ],
  )
  def k(idx_hbm, table_hbm, out_hbm, idx_v, rows_v, t_v, gsem, psem):
    wid = lax.axis_index("s") * NC + lax.axis_index("c")
    g0 = wid * j_per_w

    def blk_of(j):
      g = g0 + j
      f = g // nbh
      bh = g - f * nbh
      return f, bh

    def start_gather(j, slot):
      return pltpu.async_copy(
          table_hbm.at[idx_v.at[pl.ds(j * BLK, BLK)]],
          rows_v.at[slot], gsem)

    def wait_gather(j, slot):
      pltpu.make_async_copy(
          table_hbm.at[idx_v.at[pl.ds(j * BLK, BLK)]],
          rows_v.at[slot], gsem).wait()

    def start_puts(j, slot):
      f, bh = blk_of(j)
      for si in range(NSLAB):
        pltpu.async_copy(
            t_v.at[slot, pl.ds(si * 8, 8)], out_hbm.at[f, si, bh], psem)

    def wait_puts(j, slot):
      f, bh = blk_of(j)
      for si in range(NSLAB):
        pltpu.make_async_copy(
            t_v.at[slot, pl.ds(si * 8, 8)], out_hbm.at[f, si, bh],
            psem).wait()

    lanes = lax.iota(jnp.int32, 16)

    def transpose(slot):
      src = rows_v.at[slot]

      @plsc.parallel_loop(0, D * (BLK // 16), unroll=8)
      def _(i):
        c = lax.rem(i, D)
        chunk = lax.div(i, D)
        bl = lanes + chunk * 16
        cvec = jnp.full((16,), c, jnp.int32)
        vec = plsc.load_gather(src, [bl, cvec])
        t_v[slot, c, pl.ds(chunk * 16, 16)] = vec

    pltpu.sync_copy(idx_hbm.at[pl.ds(g0 * BLK, j_per_w * BLK)], idx_v)
    start_gather(0, 0)

    def body(j, _):
      s = lax.rem(j, 2)
      p = 1 - s
      wait_gather(j - 1, p)
      start_gather(j, s)

      @pl.when(j >= 2)
      def _():
        wait_puts(j - 2, p)

      transpose(p)
      start_puts(j - 1, p)
      return 0

    lax.fori_loop(1, j_per_w, body, 0)

    last = j_per_w - 1
    sl = last % 2
    wait_gather(last, sl)
    transpose(sl)
    start_puts(last, sl)
    wait_puts(last - 1, 1 - sl)
    wait_puts(last, sl)

  return k


def kernel(x, table):
  Bt, F = x.shape
  V, d = table.shape
  assert d == D and Bt % BLK == 0
  idx = jnp.swapaxes(x, 0, 1).reshape(F * Bt).astype(jnp.int32)
  out5 = _make_gather(F, Bt, V)(idx, table)
  return out5.transpose(2, 4, 0, 1, 3).reshape(Bt, F, D)


# speedup vs baseline: 1.4558x; 1.1740x over previous
import functools

import jax
import jax.numpy as jnp
from jax import lax
from jax.experimental import pallas as pl
from jax.experimental.pallas import tpu as pltpu
from jax.experimental.pallas import tpu_sc as plsc

D = 32
BLK = 128
NC, NS = 2, 16
NW = NC * NS
NSLAB = D // 8


@functools.lru_cache(maxsize=None)
def _make_gather(F: int, Bt: int, V: int):
  n_blocks = F * (Bt // BLK)
  assert n_blocks % NW == 0
  j_per_w = n_blocks // NW
  nbh = Bt // BLK
  mesh = plsc.VectorSubcoreMesh(core_axis_name="c", subcore_axis_name="s")

  @functools.partial(
      pl.kernel,
      out_type=jax.ShapeDtypeStruct((F, NSLAB, nbh, 8, BLK), jnp.float32),
      mesh=mesh,
      compiler_params=pltpu.CompilerParams(
          use_tc_tiling_on_sc=False, needs_layout_passes=False),
      scratch_types=[
          pltpu.VMEM((j_per_w * BLK,), jnp.int32),
          pltpu.VMEM((2, BLK, D), jnp.float32),
          pltpu.VMEM((2, D, BLK), jnp.float32),
          pltpu.SemaphoreType.DMA,
          pltpu.SemaphoreType.DMA,
      ],
  )
  def k(idx_hbm, table_hbm, out_hbm, idx_v, rows_v, t_v, gsem, psem):
    wid = lax.axis_index("s") * NC + lax.axis_index("c")
    g0 = wid * j_per_w

    def blk_of(j):
      g = g0 + j
      f = g // nbh
      bh = g - f * nbh
      return f, bh

    def start_gather(j, slot):
      return pltpu.async_copy(
          table_hbm.at[idx_v.at[pl.ds(j * BLK, BLK)]],
          rows_v.at[slot], gsem)

    def wait_gather(j, slot):
      pltpu.make_async_copy(
          table_hbm.at[idx_v.at[pl.ds(j * BLK, BLK)]],
          rows_v.at[slot], gsem).wait()

    def start_puts(j, slot):
      f, bh = blk_of(j)
      for si in range(NSLAB):
        pltpu.async_copy(
            t_v.at[slot, pl.ds(si * 8, 8)], out_hbm.at[f, si, bh], psem)

    def wait_puts(j, slot):
      f, bh = blk_of(j)
      for si in range(NSLAB):
        pltpu.make_async_copy(
            t_v.at[slot, pl.ds(si * 8, 8)], out_hbm.at[f, si, bh],
            psem).wait()

    lanes = lax.iota(jnp.int32, 16)

    def transpose(slot):
      src = rows_v.at[slot]

      @plsc.parallel_loop(0, D * (BLK // 16), unroll=8)
      def _(i):
        k = lax.rem(i, D)
        chunk = lax.div(i, D)
        bl = lanes + chunk * 16
        cvec = lax.bitwise_and(lanes + k, D - 1)
        vec = plsc.load_gather(src, [bl, cvec])
        plsc.store_scatter(t_v.at[slot], [cvec, bl], vec)

    pltpu.sync_copy(idx_hbm.at[pl.ds(g0 * BLK, j_per_w * BLK)], idx_v)
    start_gather(0, 0)

    def body(j, _):
      s = lax.rem(j, 2)
      p = 1 - s
      wait_gather(j - 1, p)
      start_gather(j, s)

      @pl.when(j >= 2)
      def _():
        wait_puts(j - 2, p)

      transpose(p)
      start_puts(j - 1, p)
      return 0

    lax.fori_loop(1, j_per_w, body, 0)

    last = j_per_w - 1
    sl = last % 2
    wait_gather(last, sl)
    transpose(sl)
    start_puts(last, sl)
    wait_puts(last - 1, 1 - sl)
    wait_puts(last, sl)

  return k


def kernel(x, table):
  Bt, F = x.shape
  V, d = table.shape
  assert d == D and Bt % BLK == 0
  idx = jnp.swapaxes(x, 0, 1).reshape(F * Bt).astype(jnp.int32)
  out5 = _make_gather(F, Bt, V)(idx, table)
  return out5.transpose(2, 4, 0, 1, 3).reshape(Bt, F, D)


# speedup vs baseline: 3.0636x; 2.1045x over previous
import functools

import jax
import jax.numpy as jnp
from jax import lax
from jax.experimental import pallas as pl
from jax.experimental.pallas import tpu as pltpu
from jax.experimental.pallas import tpu_sc as plsc

D = 32
BLK = 128
NC, NS = 2, 16
NW = NC * NS
NSLAB = D // 8


@functools.lru_cache(maxsize=None)
def _make_convert(V: int):
  nq_full = V // BLK
  tail = V - nq_full * BLK
  assert tail % 8 == 0
  mesh = plsc.VectorSubcoreMesh(core_axis_name="c", subcore_axis_name="s")

  @functools.partial(
      pl.kernel,
      out_type=jax.ShapeDtypeStruct((V * D // 128, 128), jnp.float32),
      mesh=mesh,
      compiler_params=pltpu.CompilerParams(
          use_tc_tiling_on_sc=True, needs_layout_passes=False),
      scratch_types=[
          pltpu.VMEM((2, NSLAB, 8, BLK), jnp.float32),
          pltpu.VMEM((2, D, BLK), jnp.float32),
          pltpu.SemaphoreType.DMA,
          pltpu.SemaphoreType.DMA,
      ],
  )
  def k(tt_hbm, tail_hbm, out_hbm, in_v, ob_v, gsem, psem):
    wid = lax.axis_index("s") * NC + lax.axis_index("c")
    lanes = lax.iota(jnp.int32, 16)
    nq_w = (nq_full + NW - 1) // NW

    def q_of(m):
      return wid + NW * m

    def start_in(q, slot):
      off = pl.multiple_of(q * BLK, BLK)
      for s in range(NSLAB):
        pltpu.async_copy(
            tt_hbm.at[pl.ds(s * 8, 8), pl.ds(off, BLK)],
            in_v.at[slot, s], gsem)

    def wait_in(q, slot):
      off = pl.multiple_of(q * BLK, BLK)
      for s in range(NSLAB):
        pltpu.make_async_copy(
            tt_hbm.at[pl.ds(s * 8, 8), pl.ds(off, BLK)],
            in_v.at[slot, s], gsem).wait()

    def start_out(q, slot):
      pltpu.async_copy(
          ob_v.at[slot], out_hbm.at[pl.ds(pl.multiple_of(q * D, D), D)],
          psem)

    def wait_out(q, slot):
      pltpu.make_async_copy(
          ob_v.at[slot], out_hbm.at[pl.ds(pl.multiple_of(q * D, D), D)],
          psem).wait()

    def transpose(slot):
      src = in_v.at[slot]

      @plsc.parallel_loop(0, D * (BLK // 16), unroll=8)
      def _(i):
        kk = lax.rem(i, D)
        m0 = lax.div(i, D) * 16
        m = m0 + lanes
        rw = lax.bitwise_and(lanes + kk, D - 1)
        vec = plsc.load_gather(
            src, [lax.div(rw, 8), lax.rem(rw, 8), m])
        dj = lax.div(m, 4)
        dw = lax.shift_left(lax.rem(m, 4), 5) + rw
        plsc.store_scatter(ob_v.at[slot], [dj, dw], vec)

    @pl.when(q_of(0) < nq_full)
    def _():
      start_in(q_of(0), 0)

    def body(m, _):
      q = q_of(m)
      qp = q - NW
      s = lax.rem(m, 2)
      p = 1 - s

      @pl.when(q < nq_full)
      def _():
        start_in(q, s)

      wait_in(qp, p)

      @pl.when(m >= 2)
      def _():
        wait_out(q_of(m - 2), p)

      transpose(p)
      start_out(qp, p)
      return 0

    def tail_m(m, last_q):
      s = lax.rem(m, 2)
      wait_in(last_q, s)
      transpose(s)
      start_out(last_q, s)

    n_own = (nq_full - wid + NW - 1) // NW
    lax.fori_loop(1, n_own, body, 0)
    tail_m(n_own - 1, q_of(n_own - 1))
    wait_out(q_of(n_own - 2), lax.rem(n_own, 2))
    wait_out(q_of(n_own - 1), lax.rem(n_own - 1, 2))

    @pl.when(wid == 0)
    def _():
      nr = tail * D // 128
      pltpu.sync_copy(tail_hbm, ob_v.at[0, pl.ds(0, nr)])
      pltpu.sync_copy(
          ob_v.at[0, pl.ds(0, nr)],
          out_hbm.at[pl.ds(nq_full * D, nr)])

  return k


@functools.lru_cache(maxsize=None)
def _make_gather(F: int, Bt: int, V: int):
  n_blocks = F * (Bt // BLK)
  assert n_blocks % NW == 0
  j_per_w = n_blocks // NW
  nbh = Bt // BLK
  nidx = j_per_w * BLK
  mesh = plsc.VectorSubcoreMesh(core_axis_name="c", subcore_axis_name="s")

  @functools.partial(
      pl.kernel,
      out_type=jax.ShapeDtypeStruct((F, NSLAB, nbh, 8, BLK), jnp.float32),
      mesh=mesh,
      compiler_params=pltpu.CompilerParams(
          use_tc_tiling_on_sc=False, needs_layout_passes=False),
      scratch_types=[
          pltpu.VMEM((nidx,), jnp.int32),
          pltpu.VMEM((2, BLK, D), jnp.float32),
          pltpu.VMEM((2, D, BLK), jnp.float32),
          pltpu.SemaphoreType.DMA,
          pltpu.SemaphoreType.DMA,
      ],
  )
  def k(idx_hbm, table_hbm, out_hbm, idx_v, rows_v, t_v, gsem, psem):
    wid = lax.axis_index("s") * NC + lax.axis_index("c")
    g0 = wid * j_per_w
    lanes = lax.iota(jnp.int32, 16)

    def blk_of(j):
      g = g0 + j
      f = g // nbh
      bh = g - f * nbh
      return f, bh

    def start_gather(j, slot):
      return pltpu.async_copy(
          table_hbm.at[idx_v.at[pl.ds(j * BLK, BLK)]],
          rows_v.at[slot], gsem)

    def wait_gather(j, slot):
      pltpu.make_async_copy(
          table_hbm.at[idx_v.at[pl.ds(j * BLK, BLK)]],
          rows_v.at[slot], gsem).wait()

    def start_puts(j, slot):
      f, bh = blk_of(j)
      for si in range(NSLAB):
        pltpu.async_copy(
            t_v.at[slot, pl.ds(si * 8, 8)], out_hbm.at[f, si, bh], psem)

    def wait_puts(j, slot):
      f, bh = blk_of(j)
      for si in range(NSLAB):
        pltpu.make_async_copy(
            t_v.at[slot, pl.ds(si * 8, 8)], out_hbm.at[f, si, bh],
            psem).wait()

    def transpose(slot):
      src = rows_v.at[slot]

      @plsc.parallel_loop(0, D * (BLK // 16), unroll=8)
      def _(i):
        kk = lax.rem(i, D)
        chunk = lax.div(i, D)
        bl = lanes + chunk * 16
        cvec = lax.bitwise_and(lanes + kk, D - 1)
        vec = plsc.load_gather(src, [bl, cvec])
        plsc.store_scatter(t_v.at[slot], [cvec, bl], vec)

    pltpu.sync_copy(idx_hbm.at[pl.ds(g0 * BLK, nidx)], idx_v)
    start_gather(0, 0)

    def body(j, _):
      s = lax.rem(j, 2)
      p = 1 - s
      wait_gather(j - 1, p)
      start_gather(j, s)

      @pl.when(j >= 2)
      def _():
        wait_puts(j - 2, p)

      transpose(p)
      start_puts(j - 1, p)
      return 0

    lax.fori_loop(1, j_per_w, body, 0)

    last = j_per_w - 1
    sl = last % 2
    wait_gather(last, sl)
    transpose(sl)
    start_puts(last, sl)
    wait_puts(last - 1, 1 - sl)
    wait_puts(last, sl)

  return k


def kernel(x, table):
  Bt, F = x.shape
  V, d = table.shape
  assert d == D and Bt % BLK == 0
  idx = jnp.swapaxes(x, 0, 1).reshape(F * Bt).astype(jnp.int32)
  nq_full = V // BLK
  tail16 = table[nq_full * BLK:, :].reshape(-1, 128)
  tconv = _make_convert(V)(jnp.swapaxes(table, 0, 1), tail16)
  table_rm = tconv.reshape(V, D)
  out5 = _make_gather(F, Bt, V)(idx, table_rm)
  return out5.transpose(2, 4, 0, 1, 3).reshape(Bt, F, D)


# speedup vs baseline: 3.1326x; 1.0225x over previous
import functools

import jax
import jax.numpy as jnp
from jax import lax
from jax.experimental import pallas as pl
from jax.experimental.pallas import tpu as pltpu
from jax.experimental.pallas import tpu_sc as plsc

D = 32
BLK = 128
NC, NS = 2, 16
NW = NC * NS
NSLAB = D // 8


@functools.lru_cache(maxsize=None)
def _make_convert(V: int):
  nq_full = V // BLK
  tail = V - nq_full * BLK
  assert tail % 8 == 0
  mesh = plsc.VectorSubcoreMesh(core_axis_name="c", subcore_axis_name="s")

  @functools.partial(
      pl.kernel,
      out_type=jax.ShapeDtypeStruct((V * D // 128, 128), jnp.float32),
      mesh=mesh,
      compiler_params=pltpu.CompilerParams(
          use_tc_tiling_on_sc=True, needs_layout_passes=False),
      scratch_types=[
          pltpu.VMEM((2, D, BLK), jnp.float32),
          pltpu.VMEM((2, D, BLK), jnp.float32),
          pltpu.SemaphoreType.DMA,
          pltpu.SemaphoreType.DMA,
      ],
  )
  def k(tt_hbm, tail_hbm, out_hbm, in_v, ob_v, gsem, psem):
    wid = lax.axis_index("s") * NC + lax.axis_index("c")
    lanes = lax.iota(jnp.int32, 16)
    nq_w = (nq_full + NW - 1) // NW

    def q_of(m):
      return wid + NW * m

    def start_in(q, slot):
      off = pl.multiple_of(q * BLK, BLK)
      for s in range(NSLAB):
        pltpu.async_copy(
            tt_hbm.at[pl.ds(s * 8, 8), pl.ds(off, BLK)],
            in_v.at[slot, pl.ds(s * 8, 8)], gsem)

    def wait_in(q, slot):
      off = pl.multiple_of(q * BLK, BLK)
      for s in range(NSLAB):
        pltpu.make_async_copy(
            tt_hbm.at[pl.ds(s * 8, 8), pl.ds(off, BLK)],
            in_v.at[slot, pl.ds(s * 8, 8)], gsem).wait()

    def start_out(q, slot):
      pltpu.async_copy(
          ob_v.at[slot], out_hbm.at[pl.ds(pl.multiple_of(q * D, D), D)],
          psem)

    def wait_out(q, slot):
      pltpu.make_async_copy(
          ob_v.at[slot], out_hbm.at[pl.ds(pl.multiple_of(q * D, D), D)],
          psem).wait()

    def transpose(slot):
      src = in_v.at[slot]

      @plsc.parallel_loop(0, D * (BLK // 16), unroll=8)
      def _(i):
        kk = lax.rem(i, D)
        m0 = lax.div(i, D) * 16
        m = m0 + lanes
        rw = lax.bitwise_and(lanes + kk, D - 1)
        vec = plsc.load_gather(src, [rw, m])
        dj = lax.div(m, 4)
        dw = lax.shift_left(lax.rem(m, 4), 5) + rw
        plsc.store_scatter(ob_v.at[slot], [dj, dw], vec)

    @pl.when(q_of(0) < nq_full)
    def _():
      start_in(q_of(0), 0)

    def body(m, _):
      q = q_of(m)
      qp = q - NW
      s = lax.rem(m, 2)
      p = 1 - s

      @pl.when(q < nq_full)
      def _():
        start_in(q, s)

      wait_in(qp, p)

      @pl.when(m >= 2)
      def _():
        wait_out(q_of(m - 2), p)

      transpose(p)
      start_out(qp, p)
      return 0

    def tail_m(m, last_q):
      s = lax.rem(m, 2)
      wait_in(last_q, s)
      transpose(s)
      start_out(last_q, s)

    n_own = (nq_full - wid + NW - 1) // NW
    lax.fori_loop(1, n_own, body, 0)
    tail_m(n_own - 1, q_of(n_own - 1))
    wait_out(q_of(n_own - 2), lax.rem(n_own, 2))
    wait_out(q_of(n_own - 1), lax.rem(n_own - 1, 2))

    @pl.when(wid == 0)
    def _():
      nr = tail * D // 128
      pltpu.sync_copy(tail_hbm, ob_v.at[0, pl.ds(0, nr)])
      pltpu.sync_copy(
          ob_v.at[0, pl.ds(0, nr)],
          out_hbm.at[pl.ds(nq_full * D, nr)])

  return k


@functools.lru_cache(maxsize=None)
def _make_gather(F: int, Bt: int, V: int):
  n_blocks = F * (Bt // BLK)
  assert n_blocks % NW == 0
  j_per_w = n_blocks // NW
  nbh = Bt // BLK
  nidx = j_per_w * BLK
  mesh = plsc.VectorSubcoreMesh(core_axis_name="c", subcore_axis_name="s")

  @functools.partial(
      pl.kernel,
      out_type=jax.ShapeDtypeStruct((F, NSLAB, nbh, 8, BLK), jnp.float32),
      mesh=mesh,
      compiler_params=pltpu.CompilerParams(
          use_tc_tiling_on_sc=False, needs_layout_passes=False),
      scratch_types=[
          pltpu.VMEM((nidx,), jnp.int32),
          pltpu.VMEM((2, BLK, D), jnp.float32),
          pltpu.VMEM((2, D, BLK), jnp.float32),
          pltpu.SemaphoreType.DMA,
          pltpu.SemaphoreType.DMA,
      ],
  )
  def k(idx_hbm, table_hbm, out_hbm, idx_v, rows_v, t_v, gsem, psem):
    wid = lax.axis_index("s") * NC + lax.axis_index("c")
    g0 = wid * j_per_w
    lanes = lax.iota(jnp.int32, 16)

    def blk_of(j):
      g = g0 + j
      f = g // nbh
      bh = g - f * nbh
      return f, bh

    def start_gather(j, slot):
      return pltpu.async_copy(
          table_hbm.at[idx_v.at[pl.ds(j * BLK, BLK)]],
          rows_v.at[slot], gsem)

    def wait_gather(j, slot):
      pltpu.make_async_copy(
          table_hbm.at[idx_v.at[pl.ds(j * BLK, BLK)]],
          rows_v.at[slot], gsem).wait()

    def start_puts(j, slot):
      f, bh = blk_of(j)
      for si in range(NSLAB):
        pltpu.async_copy(
            t_v.at[slot, pl.ds(si * 8, 8)], out_hbm.at[f, si, bh], psem)

    def wait_puts(j, slot):
      f, bh = blk_of(j)
      for si in range(NSLAB):
        pltpu.make_async_copy(
            t_v.at[slot, pl.ds(si * 8, 8)], out_hbm.at[f, si, bh],
            psem).wait()

    def transpose(slot):
      src = rows_v.at[slot]

      @plsc.parallel_loop(0, D * (BLK // 16), unroll=8)
      def _(i):
        kk = lax.rem(i, D)
        chunk = lax.div(i, D)
        bl = lanes + chunk * 16
        cvec = lax.bitwise_and(lanes + kk, D - 1)
        vec = plsc.load_gather(src, [bl, cvec])
        plsc.store_scatter(t_v.at[slot], [cvec, bl], vec)

    pltpu.sync_copy(idx_hbm.at[pl.ds(g0 * BLK, nidx)], idx_v)
    start_gather(0, 0)

    def body(j, _):
      s = lax.rem(j, 2)
      p = 1 - s
      wait_gather(j - 1, p)
      start_gather(j, s)

      @pl.when(j >= 2)
      def _():
        wait_puts(j - 2, p)

      transpose(p)
      start_puts(j - 1, p)
      return 0

    lax.fori_loop(1, j_per_w, body, 0)

    last = j_per_w - 1
    sl = last % 2
    wait_gather(last, sl)
    transpose(sl)
    start_puts(last, sl)
    wait_puts(last - 1, 1 - sl)
    wait_puts(last, sl)

  return k


def kernel(x, table):
  Bt, F = x.shape
  V, d = table.shape
  assert d == D and Bt % BLK == 0
  idx = jnp.swapaxes(x, 0, 1).reshape(F * Bt).astype(jnp.int32)
  nq_full = V // BLK
  tail16 = table[nq_full * BLK:, :].reshape(-1, 128)
  tconv = _make_convert(V)(jnp.swapaxes(table, 0, 1), tail16)
  table_rm = tconv.reshape(V, D)
  out5 = _make_gather(F, Bt, V)(idx, table_rm)
  return out5.transpose(2, 4, 0, 1, 3).reshape(Bt, F, D)


# speedup vs baseline: 3.1539x; 1.0068x over previous
import functools

import jax
import jax.numpy as jnp
from jax import lax
from jax.experimental import pallas as pl
from jax.experimental.pallas import tpu as pltpu
from jax.experimental.pallas import tpu_sc as plsc

D = 32
BLK = 128
NC, NS = 2, 16
NW = NC * NS
NSLAB = D // 8


@functools.lru_cache(maxsize=None)
def _make_convert(V: int):
  nq_full = V // BLK
  tail = V - nq_full * BLK
  assert tail % 8 == 0
  mesh = plsc.VectorSubcoreMesh(core_axis_name="c", subcore_axis_name="s")

  @functools.partial(
      pl.kernel,
      out_type=jax.ShapeDtypeStruct((V * D // 128, 128), jnp.float32),
      mesh=mesh,
      compiler_params=pltpu.CompilerParams(
          use_tc_tiling_on_sc=True, needs_layout_passes=False),
      scratch_types=[
          pltpu.VMEM((2, D, BLK), jnp.float32),
          pltpu.VMEM((2, D, BLK), jnp.float32),
          pltpu.SemaphoreType.DMA,
          pltpu.SemaphoreType.DMA,
      ],
  )
  def k(tt_hbm, tail_hbm, out_hbm, in_v, ob_v, gsem, psem):
    wid = lax.axis_index("s") * NC + lax.axis_index("c")
    lanes = lax.iota(jnp.int32, 16)
    nq_w = (nq_full + NW - 1) // NW

    def q_of(m):
      return wid + NW * m

    def start_in(q, slot):
      off = pl.multiple_of(q * BLK, BLK)
      for s in range(NSLAB):
        pltpu.async_copy(
            tt_hbm.at[pl.ds(s * 8, 8), pl.ds(off, BLK)],
            in_v.at[slot, pl.ds(s * 8, 8)], gsem)

    def wait_in(q, slot):
      off = pl.multiple_of(q * BLK, BLK)
      for s in range(NSLAB):
        pltpu.make_async_copy(
            tt_hbm.at[pl.ds(s * 8, 8), pl.ds(off, BLK)],
            in_v.at[slot, pl.ds(s * 8, 8)], gsem).wait()

    def start_out(q, slot):
      pltpu.async_copy(
          ob_v.at[slot], out_hbm.at[pl.ds(pl.multiple_of(q * D, D), D)],
          psem)

    def wait_out(q, slot):
      pltpu.make_async_copy(
          ob_v.at[slot], out_hbm.at[pl.ds(pl.multiple_of(q * D, D), D)],
          psem).wait()

    def transpose(slot):
      src = in_v.at[slot]

      @plsc.parallel_loop(0, D * (BLK // 16), unroll=16)
      def _(i):
        kk = lax.rem(i, D)
        m0 = lax.div(i, D) * 16
        m = m0 + lanes
        rw = lax.bitwise_and(lanes + kk, D - 1)
        vec = plsc.load_gather(src, [rw, m])
        dj = lax.div(m, 4)
        dw = lax.shift_left(lax.rem(m, 4), 5) + rw
        plsc.store_scatter(ob_v.at[slot], [dj, dw], vec)

    @pl.when(q_of(0) < nq_full)
    def _():
      start_in(q_of(0), 0)

    def body(m, _):
      q = q_of(m)
      qp = q - NW
      s = lax.rem(m, 2)
      p = 1 - s

      @pl.when(q < nq_full)
      def _():
        start_in(q, s)

      wait_in(qp, p)

      @pl.when(m >= 2)
      def _():
        wait_out(q_of(m - 2), p)

      transpose(p)
      start_out(qp, p)
      return 0

    def tail_m(m, last_q):
      s = lax.rem(m, 2)
      wait_in(last_q, s)
      transpose(s)
      start_out(last_q, s)

    n_own = (nq_full - wid + NW - 1) // NW
    lax.fori_loop(1, n_own, body, 0)
    tail_m(n_own - 1, q_of(n_own - 1))
    wait_out(q_of(n_own - 2), lax.rem(n_own, 2))
    wait_out(q_of(n_own - 1), lax.rem(n_own - 1, 2))

    @pl.when(wid == 0)
    def _():
      nr = tail * D // 128
      pltpu.sync_copy(tail_hbm, ob_v.at[0, pl.ds(0, nr)])
      pltpu.sync_copy(
          ob_v.at[0, pl.ds(0, nr)],
          out_hbm.at[pl.ds(nq_full * D, nr)])

  return k


@functools.lru_cache(maxsize=None)
def _make_gather(F: int, Bt: int, V: int):
  n_blocks = F * (Bt // BLK)
  assert n_blocks % NW == 0
  j_per_w = n_blocks // NW
  nbh = Bt // BLK
  nidx = j_per_w * BLK
  mesh = plsc.VectorSubcoreMesh(core_axis_name="c", subcore_axis_name="s")

  @functools.partial(
      pl.kernel,
      out_type=jax.ShapeDtypeStruct((F, NSLAB, nbh, 8, BLK), jnp.float32),
      mesh=mesh,
      compiler_params=pltpu.CompilerParams(
          use_tc_tiling_on_sc=False, needs_layout_passes=False),
      scratch_types=[
          pltpu.VMEM((nidx,), jnp.int32),
          pltpu.VMEM((2, BLK, D), jnp.float32),
          pltpu.VMEM((2, D, BLK), jnp.float32),
          pltpu.SemaphoreType.DMA,
          pltpu.SemaphoreType.DMA,
      ],
  )
  def k(idx_hbm, table_hbm, out_hbm, idx_v, rows_v, t_v, gsem, psem):
    wid = lax.axis_index("s") * NC + lax.axis_index("c")
    g0 = wid * j_per_w
    lanes = lax.iota(jnp.int32, 16)

    def blk_of(j):
      g = g0 + j
      f = g // nbh
      bh = g - f * nbh
      return f, bh

    def start_gather(j, slot):
      return pltpu.async_copy(
          table_hbm.at[idx_v.at[pl.ds(j * BLK, BLK)]],
          rows_v.at[slot], gsem)

    def wait_gather(j, slot):
      pltpu.make_async_copy(
          table_hbm.at[idx_v.at[pl.ds(j * BLK, BLK)]],
          rows_v.at[slot], gsem).wait()

    def start_puts(j, slot):
      f, bh = blk_of(j)
      for si in range(NSLAB):
        pltpu.async_copy(
            t_v.at[slot, pl.ds(si * 8, 8)], out_hbm.at[f, si, bh], psem)

    def wait_puts(j, slot):
      f, bh = blk_of(j)
      for si in range(NSLAB):
        pltpu.make_async_copy(
            t_v.at[slot, pl.ds(si * 8, 8)], out_hbm.at[f, si, bh],
            psem).wait()

    def transpose(slot):
      src = rows_v.at[slot]

      @plsc.parallel_loop(0, D * (BLK // 16), unroll=16)
      def _(i):
        kk = lax.rem(i, D)
        chunk = lax.div(i, D)
        bl = lanes + chunk * 16
        cvec = lax.bitwise_and(lanes + kk, D - 1)
        vec = plsc.load_gather(src, [bl, cvec])
        plsc.store_scatter(t_v.at[slot], [cvec, bl], vec)

    pltpu.sync_copy(idx_hbm.at[pl.ds(g0 * BLK, nidx)], idx_v)
    start_gather(0, 0)

    def body(j, _):
      s = lax.rem(j, 2)
      p = 1 - s
      wait_gather(j - 1, p)
      start_gather(j, s)

      @pl.when(j >= 2)
      def _():
        wait_puts(j - 2, p)

      transpose(p)
      start_puts(j - 1, p)
      return 0

    lax.fori_loop(1, j_per_w, body, 0)

    last = j_per_w - 1
    sl = last % 2
    wait_gather(last, sl)
    transpose(sl)
    start_puts(last, sl)
    wait_puts(last - 1, 1 - sl)
    wait_puts(last, sl)

  return k


def kernel(x, table):
  Bt, F = x.shape
  V, d = table.shape
  assert d == D and Bt % BLK == 0
  idx = jnp.swapaxes(x, 0, 1).reshape(F * Bt).astype(jnp.int32)
  nq_full = V // BLK
  tail16 = table[nq_full * BLK:, :].reshape(-1, 128)
  tconv = _make_convert(V)(jnp.swapaxes(table, 0, 1), tail16)
  table_rm = tconv.reshape(V, D)
  out5 = _make_gather(F, Bt, V)(idx, table_rm)
  return out5.transpose(2, 4, 0, 1, 3).reshape(Bt, F, D)


# speedup vs baseline: 3.1577x; 1.0012x over previous
import functools

import jax
import jax.numpy as jnp
from jax import lax
from jax.experimental import pallas as pl
from jax.experimental.pallas import tpu as pltpu
from jax.experimental.pallas import tpu_sc as plsc

D = 32
BLK = 128
NC, NS = 2, 16
NW = NC * NS
NSLAB = D // 8


@functools.lru_cache(maxsize=None)
def _make_convert(V: int):
  nq_full = V // BLK
  tail = V - nq_full * BLK
  assert tail % 8 == 0
  mesh = plsc.VectorSubcoreMesh(core_axis_name="c", subcore_axis_name="s")

  @functools.partial(
      pl.kernel,
      out_type=jax.ShapeDtypeStruct((V * D // 128, 128), jnp.float32),
      mesh=mesh,
      compiler_params=pltpu.CompilerParams(
          use_tc_tiling_on_sc=True, needs_layout_passes=False),
      scratch_types=[
          pltpu.VMEM((2, D, BLK), jnp.float32),
          pltpu.VMEM((2, D, BLK), jnp.float32),
          pltpu.SemaphoreType.DMA,
          pltpu.SemaphoreType.DMA,
      ],
  )
  def k(tt_hbm, tail_hbm, out_hbm, in_v, ob_v, gsem, psem):
    wid = lax.axis_index("s") * NC + lax.axis_index("c")
    lanes = lax.iota(jnp.int32, 16)

    def q_of(m):
      return wid + NW * m

    def start_in(q, slot):
      off = pl.multiple_of(q * BLK, BLK)
      for s in range(NSLAB):
        pltpu.async_copy(
            tt_hbm.at[pl.ds(s * 8, 8), pl.ds(off, BLK)],
            in_v.at[slot, pl.ds(s * 8, 8)], gsem)

    def wait_in(q, slot):
      off = pl.multiple_of(q * BLK, BLK)
      for s in range(NSLAB):
        pltpu.make_async_copy(
            tt_hbm.at[pl.ds(s * 8, 8), pl.ds(off, BLK)],
            in_v.at[slot, pl.ds(s * 8, 8)], gsem).wait()

    def start_out(q, slot):
      pltpu.async_copy(
          ob_v.at[slot], out_hbm.at[pl.ds(pl.multiple_of(q * D, D), D)],
          psem)

    def wait_out(q, slot):
      pltpu.make_async_copy(
          ob_v.at[slot], out_hbm.at[pl.ds(pl.multiple_of(q * D, D), D)],
          psem).wait()

    def transpose(slot):
      src = in_v.at[slot]

      @plsc.parallel_loop(0, D * (BLK // 16), unroll=16)
      def _(i):
        kk = lax.rem(i, D)
        m0 = lax.div(i, D) * 16
        m = m0 + lanes
        rw = lax.bitwise_and(lanes + kk, D - 1)
        vec = plsc.load_gather(src, [rw, m])
        dj = lax.div(m, 4)
        dw = lax.shift_left(lax.rem(m, 4), 5) + rw
        plsc.store_scatter(ob_v.at[slot], [dj, dw], vec)

    @pl.when(q_of(0) < nq_full)
    def _():
      start_in(q_of(0), 0)

    def body(m, _):
      q = q_of(m)
      qp = q - NW
      s = lax.rem(m, 2)
      p = 1 - s

      @pl.when(q < nq_full)
      def _():
        start_in(q, s)

      wait_in(qp, p)

      @pl.when(m >= 2)
      def _():
        wait_out(q_of(m - 2), p)

      transpose(p)
      start_out(qp, p)
      return 0

    def tail_m(m, last_q):
      s = lax.rem(m, 2)
      wait_in(last_q, s)
      transpose(s)
      start_out(last_q, s)

    n_own = (nq_full - wid + NW - 1) // NW
    lax.fori_loop(1, n_own, body, 0)
    tail_m(n_own - 1, q_of(n_own - 1))
    wait_out(q_of(n_own - 2), lax.rem(n_own, 2))
    wait_out(q_of(n_own - 1), lax.rem(n_own - 1, 2))

    @pl.when(wid == 0)
    def _():
      nr = tail * D // 128
      pltpu.sync_copy(tail_hbm, ob_v.at[0, pl.ds(0, nr)])
      pltpu.sync_copy(
          ob_v.at[0, pl.ds(0, nr)],
          out_hbm.at[pl.ds(nq_full * D, nr)])

  return k


@functools.lru_cache(maxsize=None)
def _make_gather(F: int, Bt: int, V: int):
  n_blocks = F * (Bt // BLK)
  assert n_blocks % NW == 0
  j_per_w = n_blocks // NW
  nbh = Bt // BLK
  nidx = j_per_w * BLK
  mesh = plsc.VectorSubcoreMesh(core_axis_name="c", subcore_axis_name="s")

  @functools.partial(
      pl.kernel,
      out_type=jax.ShapeDtypeStruct((F, NSLAB, nbh, 8, BLK), jnp.float32),
      mesh=mesh,
      compiler_params=pltpu.CompilerParams(
          use_tc_tiling_on_sc=False, needs_layout_passes=False),
      scratch_types=[
          pltpu.VMEM((nidx,), jnp.int32),
          pltpu.VMEM((2, BLK, D), jnp.float32),
          pltpu.VMEM((2, D, BLK), jnp.float32),
          pltpu.SemaphoreType.DMA,
          pltpu.SemaphoreType.DMA,
      ],
  )
  def k(idx_hbm, table_hbm, out_hbm, idx_v, rows_v, t_v, gsem, psem):
    wid = lax.axis_index("s") * NC + lax.axis_index("c")
    g0 = wid * j_per_w
    lanes = lax.iota(jnp.int32, 16)

    def blk_of(j):
      g = g0 + j
      f = g // nbh
      bh = g - f * nbh
      return f, bh

    def start_gather(j, slot):
      return pltpu.async_copy(
          table_hbm.at[idx_v.at[pl.ds(j * BLK, BLK)]],
          rows_v.at[slot], gsem)

    def wait_gather(j, slot):
      pltpu.make_async_copy(
          table_hbm.at[idx_v.at[pl.ds(j * BLK, BLK)]],
          rows_v.at[slot], gsem).wait()

    def start_puts(j, slot):
      f, bh = blk_of(j)
      for si in range(NSLAB):
        pltpu.async_copy(
            t_v.at[slot, pl.ds(si * 8, 8)], out_hbm.at[f, si, bh], psem)

    def wait_puts(j, slot):
      f, bh = blk_of(j)
      for si in range(NSLAB):
        pltpu.make_async_copy(
            t_v.at[slot, pl.ds(si * 8, 8)], out_hbm.at[f, si, bh],
            psem).wait()

    def transpose(slot):
      src = rows_v.at[slot]

      @plsc.parallel_loop(0, D * (BLK // 16), unroll=16)
      def _(i):
        kk = lax.rem(i, D)
        chunk = lax.div(i, D)
        bl = lanes + chunk * 16
        cvec = lax.bitwise_and(lanes + kk, D - 1)
        vec = plsc.load_gather(src, [bl, cvec])
        plsc.store_scatter(t_v.at[slot], [cvec, bl], vec)

    pltpu.sync_copy(idx_hbm.at[pl.ds(g0 * BLK, nidx)], idx_v)
    start_gather(0, 0)

    def body(j, _):
      s = lax.rem(j, 2)
      p = 1 - s
      wait_gather(j - 1, p)
      start_gather(j, s)

      @pl.when(j >= 2)
      def _():
        wait_puts(j - 2, p)

      transpose(p)
      start_puts(j - 1, p)
      return 0

    lax.fori_loop(1, j_per_w, body, 0)

    last = j_per_w - 1
    sl = last % 2
    wait_gather(last, sl)
    transpose(sl)
    start_puts(last, sl)
    wait_puts(last - 1, 1 - sl)
    wait_puts(last, sl)

  return k


def kernel(x, table):
  Bt, F = x.shape
  V, d = table.shape
  assert d == D and Bt % BLK == 0
  idx = jnp.swapaxes(x, 0, 1).reshape(F * Bt).astype(jnp.int32)
  nq_full = V // BLK
  tail16 = table[nq_full * BLK:, :].reshape(-1, 128)
  tconv = _make_convert(V)(jnp.swapaxes(table, 0, 1), tail16)
  table_rm = tconv.reshape(V, D)
  out5 = _make_gather(F, Bt, V)(idx, table_rm)
  return out5.transpose(2, 4, 0, 1, 3).reshape(Bt, F, D)
